# Initial kernel scaffold; baseline (speedup 1.0000x reference)
import jax
import jax.numpy as jnp
from jax.experimental import pallas as pl


def kernel(map_x, label_to_features, x_idx, z_idx, labels):
    raise NotImplementedError("write your pallas kernel here")


# speedup vs baseline: 9.6140x; 9.6140x over previous
import functools

import jax
import jax.numpy as jnp
from jax import lax
from jax.experimental import pallas as pl
from jax.experimental.pallas import tpu as pltpu
from jax.experimental.pallas import tpu_sc as plsc

_S = 64
_CLASSES = 16
_M = _S * _S * _CLASSES
_N = 307200

_NC = 2
_NS = 16
_NW = _NC * _NS
_CHUNK = _N // _NW
_ROWS = _CHUNK // 128
_MSLICE = _M // _NS

@functools.cache
def _make_sc_flags():
    mesh = plsc.VectorSubcoreMesh(
        core_axis_name="c", subcore_axis_name="s",
        num_cores=_NC, num_subcores=_NS,
    )
    return pl.kernel(
        _sc_flags_body,
        out_type=jax.ShapeDtypeStruct((_NC, _M), jnp.int32),
        mesh=mesh,
        scratch_types=[
            pltpu.VMEM((_CHUNK,), jnp.int32),
            pltpu.VMEM((_CHUNK,), jnp.int32),
            pltpu.VMEM((_CHUNK,), jnp.int32),
            pltpu.VMEM((_ROWS, 128), jnp.int32),
            pltpu.VMEM((128,), jnp.int32),
            pltpu.VMEM((_MSLICE,), jnp.int32),
            pltpu.VMEM_SHARED((_M,), jnp.int32),
        ],
    )


def _sc_flags_body(x_hbm, z_hbm, l_hbm, flags_hbm,
                   x_v, z_v, l_v, idx_v, ones_v, zeros_v, shared_flags):
    cid = lax.axis_index("c")
    sid = lax.axis_index("s")
    base = (cid * _NS + sid) * _CHUNK

    pltpu.sync_copy(x_hbm.at[pl.ds(base, _CHUNK)], x_v)
    pltpu.sync_copy(z_hbm.at[pl.ds(base, _CHUNK)], z_v)
    pltpu.sync_copy(l_hbm.at[pl.ds(base, _CHUNK)], l_v)

    def _fill_const(i, _):
        ones_v[pl.ds(i * 16, 16)] = jnp.full((16,), 1, jnp.int32)
        return 0

    lax.fori_loop(0, 128 // 16, _fill_const, 0)

    def _fill_zero(i, _):
        zeros_v[pl.ds(i * 16, 16)] = jnp.full((16,), 0, jnp.int32)
        return 0

    lax.fori_loop(0, _MSLICE // 16, _fill_zero, 0)

    def _idx_row(j, _):
        def _idx_col(t, _):
            off = j * 128 + t * 16
            xv = x_v[pl.ds(off, 16)]
            zv = z_v[pl.ds(off, 16)]
            lv = l_v[pl.ds(off, 16)]
            idx_v[j, pl.ds(t * 16, 16)] = xv + zv * _S + lv * (_S * _S)
            return 0

        lax.fori_loop(0, 128 // 16, _idx_col, 0)
        return 0

    lax.fori_loop(0, _ROWS, _idx_row, 0)

    pltpu.sync_copy(zeros_v, shared_flags.at[pl.ds(sid * _MSLICE, _MSLICE)])
    plsc.subcore_barrier()

    def _scatter_row(j, _):
        pltpu.sync_copy(ones_v, shared_flags.at[idx_v.at[j]], add=True)
        return 0

    lax.fori_loop(0, _ROWS, _scatter_row, 0)
    plsc.subcore_barrier()

    pltpu.sync_copy(shared_flags.at[pl.ds(sid * _MSLICE, _MSLICE)],
                    flags_hbm.at[cid, pl.ds(sid * _MSLICE, _MSLICE)])


_R = 512
_NBLK = _M // _R


def _tc_select_body(map_ref, ltf_ref, f0_ref, f1_ref, out_ref):
    touched = (f0_ref[...] + f1_ref[...]).reshape(_R, 1) > 0
    row = ltf_ref[...].reshape(1, ltf_ref.shape[2])
    out_ref[...] = jnp.where(touched, row, map_ref[...])


def kernel(map_x, label_to_features, x_idx, z_idx, labels):
    feat = map_x.shape[1]
    flags = _make_sc_flags()(x_idx, z_idx, labels)
    f0 = flags[0].reshape(_NBLK, _R, 1)
    f1 = flags[1].reshape(_NBLK, _R, 1)
    ltf3 = label_to_features.reshape(_CLASSES, 1, feat)
    out = pl.pallas_call(
        _tc_select_body,
        grid=(_NBLK,),
        in_specs=[
            pl.BlockSpec((_R, feat), lambda i: (i, 0)),
            pl.BlockSpec((1, 1, feat), lambda i: (i // (4096 // _R), 0, 0)),
            pl.BlockSpec((1, _R, 1), lambda i: (i, 0, 0)),
            pl.BlockSpec((1, _R, 1), lambda i: (i, 0, 0)),
        ],
        out_specs=pl.BlockSpec((_R, feat), lambda i: (i, 0)),
        out_shape=jax.ShapeDtypeStruct((_M, feat), map_x.dtype),
    )(map_x, ltf3, f0, f1)
    return out


# speedup vs baseline: 11.3409x; 1.1796x over previous
import functools

import jax
import jax.numpy as jnp
from jax import lax
from jax.experimental import pallas as pl
from jax.experimental.pallas import tpu as pltpu
from jax.experimental.pallas import tpu_sc as plsc

_S = 64
_CLASSES = 16
_M = _S * _S * _CLASSES
_N = 307200

_NC = 2
_NS = 16
_NW = _NC * _NS
_CHUNK = _N // _NW
_ROWS = _CHUNK // 128
_MSLICE = _M // _NS

@functools.cache
def _make_sc_flags():
    mesh = plsc.VectorSubcoreMesh(
        core_axis_name="c", subcore_axis_name="s",
        num_cores=_NC, num_subcores=_NS,
    )
    return pl.kernel(
        _sc_flags_body,
        out_type=jax.ShapeDtypeStruct((_NC, _M), jnp.int32),
        mesh=mesh,
        scratch_types=[
            pltpu.VMEM((_CHUNK,), jnp.int32),
            pltpu.VMEM((_CHUNK,), jnp.int32),
            pltpu.VMEM((_CHUNK,), jnp.int32),
            pltpu.VMEM((_ROWS, 128), jnp.int32),
            pltpu.VMEM((128,), jnp.int32),
            pltpu.VMEM((_MSLICE,), jnp.int32),
            pltpu.VMEM_SHARED((_M,), jnp.int32),
        ],
    )


def _sc_flags_body(x_hbm, z_hbm, l_hbm, flags_hbm,
                   x_v, z_v, l_v, idx_v, ones_v, zeros_v, shared_flags):
    cid = lax.axis_index("c")
    sid = lax.axis_index("s")
    base = (cid * _NS + sid) * _CHUNK

    pltpu.sync_copy(x_hbm.at[pl.ds(base, _CHUNK)], x_v)
    pltpu.sync_copy(z_hbm.at[pl.ds(base, _CHUNK)], z_v)
    pltpu.sync_copy(l_hbm.at[pl.ds(base, _CHUNK)], l_v)

    def _fill_const(i, _):
        ones_v[pl.ds(i * 16, 16)] = jnp.full((16,), 1, jnp.int32)
        return 0

    lax.fori_loop(0, 128 // 16, _fill_const, 0)

    def _fill_zero(i, _):
        zeros_v[pl.ds(i * 16, 16)] = jnp.full((16,), 0, jnp.int32)
        return 0

    lax.fori_loop(0, _MSLICE // 16, _fill_zero, 0)

    def _idx_row(j, _):
        def _idx_col(t, _):
            off = j * 128 + t * 16
            xv = x_v[pl.ds(off, 16)]
            zv = z_v[pl.ds(off, 16)]
            lv = l_v[pl.ds(off, 16)]
            idx_v[j, pl.ds(t * 16, 16)] = xv + zv * _S + lv * (_S * _S)
            return 0

        lax.fori_loop(0, 128 // 16, _idx_col, 0)
        return 0

    lax.fori_loop(0, _ROWS, _idx_row, 0)

    pltpu.sync_copy(zeros_v, shared_flags.at[pl.ds(sid * _MSLICE, _MSLICE)])
    plsc.subcore_barrier()

    def _scatter_row(j, _):
        pltpu.sync_copy(ones_v, shared_flags.at[idx_v.at[j]], add=True)
        return 0

    lax.fori_loop(0, _ROWS, _scatter_row, 0)
    plsc.subcore_barrier()

    pltpu.sync_copy(shared_flags.at[pl.ds(sid * _MSLICE, _MSLICE)],
                    flags_hbm.at[cid, pl.ds(sid * _MSLICE, _MSLICE)])


_R = 2048
_NBLK = _M // _R


def _tc_select_body(map_ref, ltf_ref, flags_ref, out_ref):
    counts = flags_ref[0:1, :] + flags_ref[1:2, :]
    touched = counts.reshape(_R, 1) > 0
    row = ltf_ref[...].reshape(1, ltf_ref.shape[2])
    out_ref[...] = jnp.where(touched, row, map_ref[...])


def kernel(map_x, label_to_features, x_idx, z_idx, labels):
    feat = map_x.shape[1]
    flags = _make_sc_flags()(x_idx, z_idx, labels)
    ltf3 = label_to_features.reshape(_CLASSES, 1, feat)
    out = pl.pallas_call(
        _tc_select_body,
        grid=(_NBLK,),
        in_specs=[
            pl.BlockSpec((_R, feat), lambda i: (i, 0)),
            pl.BlockSpec((1, 1, feat), lambda i: (i // (4096 // _R), 0, 0)),
            pl.BlockSpec((2, _R), lambda i: (0, i)),
        ],
        out_specs=pl.BlockSpec((_R, feat), lambda i: (i, 0)),
        out_shape=jax.ShapeDtypeStruct((_M, feat), map_x.dtype),
    )(map_x, ltf3, flags)
    return out


# speedup vs baseline: 12.2558x; 1.0807x over previous
import functools

import jax
import jax.numpy as jnp
from jax import lax
from jax.experimental import pallas as pl
from jax.experimental.pallas import tpu as pltpu
from jax.experimental.pallas import tpu_sc as plsc

_S = 64
_CLASSES = 16
_M = _S * _S * _CLASSES
_N = 307200

_NC = 2
_NS = 16
_TCH = _N // _NS
_TROWS = _TCH // 128
_H = _M // _NC
_HS = _H // _NS
_NDUMP = _NS * 16
_SP = _H + _NDUMP
_SP2 = _SP + 4096
_ZS = _SP2 // _NS

_G = 16
_NGRP = _M // _G


@functools.cache
def _make_sc_flags():
    mesh = plsc.VectorSubcoreMesh(
        core_axis_name="c", subcore_axis_name="s",
        num_cores=_NC, num_subcores=_NS,
    )
    return pl.kernel(
        _sc_flags_body,
        out_type=(
            jax.ShapeDtypeStruct((_M,), jnp.int32),
            jax.ShapeDtypeStruct((_NGRP,), jnp.int32),
        ),
        mesh=mesh,
        scratch_types=[
            pltpu.VMEM((_TCH,), jnp.int32),
            pltpu.VMEM((_TCH,), jnp.int32),
            pltpu.VMEM((_TCH,), jnp.int32),
            pltpu.VMEM((_TROWS, 128), jnp.int32),
            pltpu.VMEM((128,), jnp.int32),
            pltpu.VMEM((_ZS,), jnp.int32),
            pltpu.VMEM((_HS,), jnp.int32),
            pltpu.VMEM((16, 128), jnp.int32),
            pltpu.VMEM((16, 128), jnp.int32),
            pltpu.VMEM((128,), jnp.int32),
            pltpu.VMEM_SHARED((_SP2,), jnp.int32),
        ],
    )


def _sc_flags_body(x_hbm, z_hbm, l_hbm, flags_hbm, needy_hbm,
                   x_v, z_v, l_v, idx_v, ones_v, zeros_v, counts_v, u_v,
                   gidx_v, needy_v, shared_counts):
    cid = lax.axis_index("c")
    sid = lax.axis_index("s")
    base = sid * _TCH
    hbase = cid * _H
    iota16 = jnp.arange(16, dtype=jnp.int32)
    dump_vec = _H + sid * 16 + iota16

    pltpu.sync_copy(x_hbm.at[pl.ds(base, _TCH)], x_v)
    pltpu.sync_copy(z_hbm.at[pl.ds(base, _TCH)], z_v)
    pltpu.sync_copy(l_hbm.at[pl.ds(base, _TCH)], l_v)

    def _fill_ones(i, _):
        ones_v[pl.ds(i * 16, 16)] = jnp.full((16,), 1, jnp.int32)
        return 0

    lax.fori_loop(0, 128 // 16, _fill_ones, 0)

    def _fill_zero(i, _):
        zeros_v[pl.ds(i * 16, 16)] = jnp.full((16,), 0, jnp.int32)
        return 0

    lax.fori_loop(0, _ZS // 16, _fill_zero, 0)

    def _idx_row(j, _):
        def _idx_col(t, _):
            off = j * 128 + t * 16
            xv = x_v[pl.ds(off, 16)]
            zv = z_v[pl.ds(off, 16)]
            lv = l_v[pl.ds(off, 16)]
            idx = xv + zv * _S + lv * (_S * _S) - hbase
            valid = (idx >= 0) & (idx < _H)
            idx_v[j, pl.ds(t * 16, 16)] = jnp.where(valid, idx, dump_vec)
            return 0

        lax.fori_loop(0, 128 // 16, _idx_col, 0)
        return 0

    lax.fori_loop(0, _TROWS, _idx_row, 0)

    pltpu.sync_copy(zeros_v, shared_counts.at[pl.ds(sid * _ZS, _ZS)])
    plsc.subcore_barrier()

    def _scatter_row(j, _):
        pltpu.sync_copy(ones_v, shared_counts.at[idx_v.at[j]], add=True)
        return 0

    lax.fori_loop(0, _TROWS, _scatter_row, 0)
    plsc.subcore_barrier()

    pltpu.sync_copy(shared_counts.at[pl.ds(sid * _HS, _HS)], counts_v)
    pltpu.sync_copy(counts_v, flags_hbm.at[pl.ds(cid * _H + sid * _HS, _HS)])

    ones16 = jnp.full((16,), 1, jnp.int32)
    zeros16 = jnp.full((16,), 0, jnp.int32)

    def _u_row(j, _):
        def _u_col(t, _):
            c = j * 8 + t
            u_v[j, pl.ds(t * 16, 16)] = jnp.where(
                counts_v[pl.ds(c * 16, 16)] == 0, ones16, zeros16)
            gidx_v[j, pl.ds(t * 16, 16)] = (_SP + sid * 128 + c) + iota16 * 0
            return 0

        lax.fori_loop(0, 128 // 16, _u_col, 0)
        return 0

    lax.fori_loop(0, 16, _u_row, 0)

    def _gscatter(j, _):
        pltpu.sync_copy(u_v.at[j], shared_counts.at[gidx_v.at[j]], add=True)
        return 0

    lax.fori_loop(0, 16, _gscatter, 0)
    plsc.subcore_barrier()
    pltpu.sync_copy(shared_counts.at[pl.ds(_SP + sid * 128, 128)], needy_v)
    pltpu.sync_copy(
        needy_v,
        needy_hbm.at[pl.ds(cid * (_H // _G) + sid * (_HS // _G), _HS // _G)])


_R = 2048
_NBLK = _M // _R
_NGB = _R // _G


def _tc_build_body(needy_ref, flags_ref, ltf_ref, map_any, out_ref,
                   rowbuf, sems):
    i = pl.program_id(0)
    feat = out_ref.shape[1]

    def _group_copy(blk, parity, g):
        return pltpu.make_async_copy(
            map_any.at[pl.ds(blk * _R + g * _G, _G), :],
            rowbuf.at[parity, pl.ds(g * _G, _G), :],
            sems.at[parity],
        )

    def _issue(blk, parity):
        def gbody(g, _):
            @pl.when(needy_ref[blk * _NGB + g] != 0)
            def _():
                _group_copy(blk, parity, g).start()
            return 0

        lax.fori_loop(0, _NGB, gbody, 0)

    def _drain(blk, parity):
        def gbody(g, _):
            @pl.when(needy_ref[blk * _NGB + g] != 0)
            def _():
                _group_copy(blk, parity, g).wait()
            return 0

        lax.fori_loop(0, _NGB, gbody, 0)

    @pl.when(i == 0)
    def _():
        _issue(0, 0)

    @pl.when(i + 1 < _NBLK)
    def _():
        _issue(i + 1, (i + 1) % 2)

    _drain(i, i % 2)

    touched = flags_ref[...].reshape(_R, 1) > 0
    row = ltf_ref[...].reshape(1, feat)
    out_ref[...] = jnp.where(touched, row, rowbuf[i % 2])


def kernel(map_x, label_to_features, x_idx, z_idx, labels):
    feat = map_x.shape[1]
    flags, needy = _make_sc_flags()(x_idx, z_idx, labels)
    ltf3 = label_to_features.reshape(_CLASSES, 1, feat)
    grid_spec = pltpu.PrefetchScalarGridSpec(
        num_scalar_prefetch=1,
        grid=(_NBLK,),
        in_specs=[
            pl.BlockSpec((_R,), lambda i, needy: (i,)),
            pl.BlockSpec((1, 1, feat), lambda i, needy: (i // (4096 // _R), 0, 0)),
            pl.BlockSpec(memory_space=pl.ANY),
        ],
        out_specs=pl.BlockSpec((_R, feat), lambda i, needy: (i, 0)),
        scratch_shapes=[
            pltpu.VMEM((2, _R, feat), jnp.float32),
            pltpu.SemaphoreType.DMA((2,)),
        ],
    )
    out = pl.pallas_call(
        _tc_build_body,
        grid_spec=grid_spec,
        out_shape=jax.ShapeDtypeStruct((_M, feat), map_x.dtype),
    )(needy, flags, ltf3, map_x)
    return out


# speedup vs baseline: 12.6393x; 1.0313x over previous
import functools

import jax
import jax.numpy as jnp
from jax import lax
from jax.experimental import pallas as pl
from jax.experimental.pallas import tpu as pltpu
from jax.experimental.pallas import tpu_sc as plsc

_S = 64
_CLASSES = 16
_M = _S * _S * _CLASSES
_N = 307200

_NC = 2
_NS = 16
_TCH = _N // _NS
_TROWS = _TCH // 128
_H = _M // _NC
_HS = _H // _NS
_NDUMP = _NS * 16
_SP = _H + _NDUMP
_SP2 = _SP + 4096
_ZS = _SP2 // _NS

_G = 16
_NGRP = _M // _G


@functools.cache
def _make_sc_flags():
    mesh = plsc.VectorSubcoreMesh(
        core_axis_name="c", subcore_axis_name="s",
        num_cores=_NC, num_subcores=_NS,
    )
    return pl.kernel(
        _sc_flags_body,
        out_type=(
            jax.ShapeDtypeStruct((_M,), jnp.int32),
            jax.ShapeDtypeStruct((_NGRP + 128,), jnp.int32),
        ),
        mesh=mesh,
        scratch_types=[
            pltpu.VMEM((_TCH,), jnp.int32),
            pltpu.VMEM((_TCH,), jnp.int32),
            pltpu.VMEM((_TCH,), jnp.int32),
            pltpu.VMEM((_TROWS, 128), jnp.int32),
            pltpu.VMEM((128,), jnp.int32),
            pltpu.VMEM((_ZS,), jnp.int32),
            pltpu.VMEM((_HS,), jnp.int32),
            pltpu.VMEM((16, 128), jnp.int32),
            pltpu.VMEM((16, 128), jnp.int32),
            pltpu.VMEM((128,), jnp.int32),
            pltpu.VMEM_SHARED((_SP2,), jnp.int32),
            pltpu.SemaphoreType.DMA,
        ],
    )


def _sc_flags_body(x_hbm, z_hbm, l_hbm, flags_hbm, needy_hbm,
                   x_v, z_v, l_v, idx_v, ones_v, zeros_v, counts_v, u_v,
                   gidx_v, needy_v, shared_counts, dma_sem):
    cid = lax.axis_index("c")
    sid = lax.axis_index("s")
    base = sid * _TCH
    hbase = cid * _H
    iota16 = jnp.arange(16, dtype=jnp.int32)
    dump_vec = _H + sid * 16 + iota16

    pltpu.sync_copy(x_hbm.at[pl.ds(base, _TCH)], x_v)
    pltpu.sync_copy(z_hbm.at[pl.ds(base, _TCH)], z_v)
    pltpu.sync_copy(l_hbm.at[pl.ds(base, _TCH)], l_v)

    def _fill_ones(i, _):
        ones_v[pl.ds(i * 16, 16)] = jnp.full((16,), 1, jnp.int32)
        return 0

    lax.fori_loop(0, 128 // 16, _fill_ones, 0)

    def _fill_zero(i, _):
        zeros_v[pl.ds(i * 16, 16)] = jnp.full((16,), 0, jnp.int32)
        return 0

    lax.fori_loop(0, _ZS // 16, _fill_zero, 0)

    def _idx_row(j, _):
        def _idx_col(t, _):
            off = j * 128 + t * 16
            xv = x_v[pl.ds(off, 16)]
            zv = z_v[pl.ds(off, 16)]
            lv = l_v[pl.ds(off, 16)]
            idx = xv + zv * _S + lv * (_S * _S) - hbase
            valid = (idx >= 0) & (idx < _H)
            idx_v[j, pl.ds(t * 16, 16)] = jnp.where(valid, idx, dump_vec)
            return 0

        lax.fori_loop(0, 128 // 16, _idx_col, 0)
        return 0

    lax.fori_loop(0, _TROWS, _idx_row, 0)

    pltpu.sync_copy(zeros_v, shared_counts.at[pl.ds(sid * _ZS, _ZS)])
    plsc.subcore_barrier()

    def _scatter_chunk(j2, _):
        def _fire(t, _):
            pltpu.async_copy(
                ones_v, shared_counts.at[idx_v.at[j2 * 10 + t]], dma_sem,
                add=True)
            return 0

        def _drain(t, _):
            pltpu.make_async_copy(
                ones_v, shared_counts.at[idx_v.at[j2 * 10 + t]],
                dma_sem).wait()
            return 0

        lax.fori_loop(0, 10, _fire, 0)
        lax.fori_loop(0, 10, _drain, 0)
        return 0

    lax.fori_loop(0, _TROWS // 10, _scatter_chunk, 0)
    plsc.subcore_barrier()

    pltpu.sync_copy(shared_counts.at[pl.ds(sid * _HS, _HS)], counts_v)
    pltpu.sync_copy(counts_v, flags_hbm.at[pl.ds(cid * _H + sid * _HS, _HS)])

    ones16 = jnp.full((16,), 1, jnp.int32)
    zeros16 = jnp.full((16,), 0, jnp.int32)

    def _u_row(j, _):
        def _u_col(t, _):
            c = j * 8 + t
            u_v[j, pl.ds(t * 16, 16)] = jnp.where(
                counts_v[pl.ds(c * 16, 16)] == 0, ones16, zeros16)
            gidx_v[j, pl.ds(t * 16, 16)] = (_SP + sid * 128 + c) + iota16 * 0
            return 0

        lax.fori_loop(0, 128 // 16, _u_col, 0)
        return 0

    lax.fori_loop(0, 16, _u_row, 0)

    def _gscatter(j, _):
        pltpu.sync_copy(u_v.at[j], shared_counts.at[gidx_v.at[j]], add=True)
        return 0

    lax.fori_loop(0, 16, _gscatter, 0)
    plsc.subcore_barrier()
    pltpu.sync_copy(shared_counts.at[pl.ds(_SP + sid * 128, 128)], needy_v)
    pltpu.sync_copy(
        needy_v,
        needy_hbm.at[pl.ds(cid * (_H // _G) + sid * (_HS // _G), _HS // _G)])


_R = 2048
_NBLK = _M // _R
_NGB = _R // _G


def _tc_build_body(needy_ref, flags_ref, ltf_ref, map_any, out_ref,
                   rowbuf, sems):
    i = pl.program_id(0)
    feat = out_ref.shape[1]

    def _group_copy(blk, parity, g):
        return pltpu.make_async_copy(
            map_any.at[pl.ds(blk * _R + g * _G, _G), :],
            rowbuf.at[parity, pl.ds(g * _G, _G), :],
            sems.at[parity],
        )

    @pl.when(i == 0)
    def _():
        def gbody0(g, _):
            @pl.when(needy_ref[g] != 0)
            def _():
                _group_copy(0, 0, g).start()
            return 0

        lax.fori_loop(0, _NGB, gbody0, 0)

    def gbody(g, _):
        @pl.when(needy_ref[i * _NGB + g] != 0)
        def _():
            _group_copy(i, i % 2, g).wait()

        @pl.when((i + 1 < _NBLK) & (needy_ref[(i + 1) * _NGB + g] != 0))
        def _():
            _group_copy(i + 1, (i + 1) % 2, g).start()

        return 0

    lax.fori_loop(0, _NGB, gbody, 0, unroll=2)

    touched = flags_ref[...].reshape(_R, 1) > 0
    row = ltf_ref[...].reshape(1, feat)
    out_ref[...] = jnp.where(touched, row, rowbuf[i % 2])


def kernel(map_x, label_to_features, x_idx, z_idx, labels):
    feat = map_x.shape[1]
    flags, needy = _make_sc_flags()(x_idx, z_idx, labels)
    ltf3 = label_to_features.reshape(_CLASSES, 1, feat)
    grid_spec = pltpu.PrefetchScalarGridSpec(
        num_scalar_prefetch=1,
        grid=(_NBLK,),
        in_specs=[
            pl.BlockSpec((_R,), lambda i, needy: (i,)),
            pl.BlockSpec((1, 1, feat), lambda i, needy: (i // (4096 // _R), 0, 0)),
            pl.BlockSpec(memory_space=pl.ANY),
        ],
        out_specs=pl.BlockSpec((_R, feat), lambda i, needy: (i, 0)),
        scratch_shapes=[
            pltpu.VMEM((2, _R, feat), jnp.float32),
            pltpu.SemaphoreType.DMA((2,)),
        ],
    )
    out = pl.pallas_call(
        _tc_build_body,
        grid_spec=grid_spec,
        out_shape=jax.ShapeDtypeStruct((_M, feat), map_x.dtype),
    )(needy, flags, ltf3, map_x)
    return out
